# Initial kernel scaffold; baseline (speedup 1.0000x reference)
import jax
import jax.numpy as jnp
from jax.experimental import pallas as pl


def kernel(x, pos, batch):
    raise NotImplementedError("write your pallas kernel here")


# speedup vs baseline: 2.1576x; 2.1576x over previous
import functools

import jax
import jax.numpy as jnp
import numpy as np
from jax import lax
from jax.experimental import pallas as pl
from jax.experimental.pallas import tpu as pltpu
from jax.experimental.pallas import tpu_sc as plsc

N = 8192
NC, NS, L = 2, 16, 16
NW = NC * NS
QPW = N // NW
CHUNKS = QPW // L

T1 = np.float32(0.2 * 0.2)
T2 = np.float32(0.4 * 0.4)


def _sc_counts(xs, ys, zs, b32, off16):
    mesh = plsc.VectorSubcoreMesh(
        core_axis_name="c", subcore_axis_name="s",
        num_cores=NC, num_subcores=NS)

    @functools.partial(
        pl.kernel,
        out_type=(jax.ShapeDtypeStruct((N,), jnp.float32),
                  jax.ShapeDtypeStruct((N,), jnp.float32)),
        mesh=mesh,
        scratch_types=[
            pltpu.VMEM((N,), jnp.float32),
            pltpu.VMEM((N,), jnp.float32),
            pltpu.VMEM((N,), jnp.float32),
            pltpu.VMEM((N,), jnp.int32),
            pltpu.VMEM((L,), jnp.int32),
            pltpu.VMEM((QPW,), jnp.float32),
            pltpu.VMEM((QPW,), jnp.float32),
        ],
        compiler_params=pltpu.CompilerParams(
            use_tc_tiling_on_sc=False, needs_layout_passes=False),
    )
    def k(xs_h, ys_h, zs_h, b_h, off_h, c1_h, c2_h,
          xs_v, ys_v, zs_v, b_v, off_v, c1_v, c2_v):
        wid = lax.axis_index("s") * NC + lax.axis_index("c")
        pltpu.sync_copy(xs_h, xs_v)
        pltpu.sync_copy(ys_h, ys_v)
        pltpu.sync_copy(zs_h, zs_v)
        pltpu.sync_copy(b_h, b_v)
        pltpu.sync_copy(off_h, off_v)
        qbase = wid * QPW
        lane = lax.iota(jnp.int32, L)

        for c in range(CHUNKS):
            qidx = qbase + c * L + lane
            qx = plsc.load_gather(xs_v, [qidx])
            qy = plsc.load_gather(ys_v, [qidx])
            qz = plsc.load_gather(zs_v, [qidx])
            bq = plsc.load_gather(b_v, [qidx])
            sv = plsc.load_gather(off_v, [bq])
            ev = plsc.load_gather(off_v, [bq + 1])
            jstart = jnp.min(sv)
            jend = jnp.max(ev)

            def body(j, carry):
                a1, a2 = carry
                jv = jnp.full((L,), j, dtype=jnp.int32)
                xj = plsc.load_gather(xs_v, [jv])
                yj = plsc.load_gather(ys_v, [jv])
                zj = plsc.load_gather(zs_v, [jv])
                bj = plsc.load_gather(b_v, [jv])
                m = bj == bq
                dx = qx - xj
                dy = qy - yj
                dz = qz - zj
                d2 = dx * dx + dy * dy + dz * dz
                one = np.float32(1.0)
                zero = np.float32(0.0)
                a1 = a1 + jnp.where(m & (d2 <= T1), one, zero)
                a2 = a2 + jnp.where(m & (d2 <= T2), one, zero)
                return a1, a2

            z16 = jnp.zeros((L,), jnp.float32)
            a1, a2 = lax.fori_loop(jstart, jend, body, (z16, z16))
            c1_v[pl.ds(c * L, L)] = jnp.minimum(a1, np.float32(32.0)) * np.float32(1.0 / 32.0)
            c2_v[pl.ds(c * L, L)] = jnp.minimum(a2, np.float32(64.0)) * np.float32(1.0 / 64.0)

        pltpu.sync_copy(c1_v, c1_h.at[pl.ds(qbase, QPW)])
        pltpu.sync_copy(c2_v, c2_h.at[pl.ds(qbase, QPW)])

    return k(xs, ys, zs, b32, off16)


def kernel(x, pos, batch):
    pos = pos.astype(jnp.float32)
    xs = pos[:, 0]
    ys = pos[:, 1]
    zs = pos[:, 2]
    b32 = batch.astype(jnp.int32)
    off = jnp.searchsorted(b32, jnp.arange(9, dtype=jnp.int32)).astype(jnp.int32)
    off16 = jnp.concatenate([off, jnp.full((L - 9,), N, jnp.int32)])
    c1, c2 = _sc_counts(xs, ys, zs, b32, off16)
    feats = jnp.concatenate([x, pos, c1[:, None], c2[:, None]], axis=1)
    return feats, pos, batch


# speedup vs baseline: 2.2561x; 1.0456x over previous
import functools

import jax
import jax.numpy as jnp
import numpy as np
from jax import lax
from jax.experimental import pallas as pl
from jax.experimental.pallas import tpu as pltpu
from jax.experimental.pallas import tpu_sc as plsc

N = 8192
NC, NS, L = 2, 16, 16
NW = NC * NS
QPW = N // NW
CHUNKS = QPW // L
CPG = 4

T1 = np.float32(0.2 * 0.2)
T2 = np.float32(0.4 * 0.4)


def _sc_counts(xs, ys, zs, b32, off16):
    mesh = plsc.VectorSubcoreMesh(
        core_axis_name="c", subcore_axis_name="s",
        num_cores=NC, num_subcores=NS)

    @functools.partial(
        pl.kernel,
        out_type=(jax.ShapeDtypeStruct((N,), jnp.float32),
                  jax.ShapeDtypeStruct((N,), jnp.float32)),
        mesh=mesh,
        scratch_types=[
            pltpu.VMEM((N,), jnp.float32),
            pltpu.VMEM((N,), jnp.float32),
            pltpu.VMEM((N,), jnp.float32),
            pltpu.VMEM((N,), jnp.int32),
            pltpu.VMEM((L,), jnp.int32),
            pltpu.VMEM((QPW,), jnp.float32),
            pltpu.VMEM((QPW,), jnp.float32),
        ],
        compiler_params=pltpu.CompilerParams(
            use_tc_tiling_on_sc=False, needs_layout_passes=False),
    )
    def k(xs_h, ys_h, zs_h, b_h, off_h, c1_h, c2_h,
          xs_v, ys_v, zs_v, b_v, off_v, c1_v, c2_v):
        wid = lax.axis_index("s") * NC + lax.axis_index("c")
        pltpu.sync_copy(xs_h, xs_v)
        pltpu.sync_copy(ys_h, ys_v)
        pltpu.sync_copy(zs_h, zs_v)
        pltpu.sync_copy(b_h, b_v)
        pltpu.sync_copy(off_h, off_v)
        qbase = wid * QPW
        lane = lax.iota(jnp.int32, L)

        for sc in range(QPW // (L * CPG)):
            qx, qy, qz, bq = [], [], [], []
            jstart = jnp.int32(N)
            jend = jnp.int32(0)
            for g in range(CPG):
                qidx = qbase + (sc * CPG + g) * L + lane
                qx.append(plsc.load_gather(xs_v, [qidx]))
                qy.append(plsc.load_gather(ys_v, [qidx]))
                qz.append(plsc.load_gather(zs_v, [qidx]))
                bq.append(plsc.load_gather(b_v, [qidx]))
                sv = plsc.load_gather(off_v, [bq[g]])
                ev = plsc.load_gather(off_v, [bq[g] + 1])
                jstart = jnp.minimum(jstart, jnp.min(sv))
                jend = jnp.maximum(jend, jnp.max(ev))

            one = np.float32(1.0)
            zero = np.float32(0.0)
            z16 = jnp.zeros((L,), jnp.float32)

            def body(j, carry):
                jv = jnp.full((L,), j, dtype=jnp.int32)
                xj = plsc.load_gather(xs_v, [jv])
                yj = plsc.load_gather(ys_v, [jv])
                zj = plsc.load_gather(zs_v, [jv])
                bj = plsc.load_gather(b_v, [jv])
                out = []
                for g in range(CPG):
                    a1, a2 = carry[2 * g], carry[2 * g + 1]
                    m = bj == bq[g]
                    dx = qx[g] - xj
                    dy = qy[g] - yj
                    dz = qz[g] - zj
                    d2 = dx * dx + dy * dy + dz * dz
                    out.append(a1 + jnp.where(m & (d2 <= T1), one, zero))
                    out.append(a2 + jnp.where(m & (d2 <= T2), one, zero))
                return tuple(out)

            acc = plsc.parallel_loop(
                jstart, jend, 1, unroll=2, carry=(z16,) * (2 * CPG))(body)
            for g in range(CPG):
                o = (sc * CPG + g) * L
                c1_v[pl.ds(o, L)] = (
                    jnp.minimum(acc[2 * g], np.float32(32.0)) * np.float32(1.0 / 32.0))
                c2_v[pl.ds(o, L)] = (
                    jnp.minimum(acc[2 * g + 1], np.float32(64.0)) * np.float32(1.0 / 64.0))

        pltpu.sync_copy(c1_v, c1_h.at[pl.ds(qbase, QPW)])
        pltpu.sync_copy(c2_v, c2_h.at[pl.ds(qbase, QPW)])

    return k(xs, ys, zs, b32, off16)


def kernel(x, pos, batch):
    pos = pos.astype(jnp.float32)
    xs = pos[:, 0]
    ys = pos[:, 1]
    zs = pos[:, 2]
    b32 = batch.astype(jnp.int32)
    off = jnp.searchsorted(b32, jnp.arange(9, dtype=jnp.int32)).astype(jnp.int32)
    off16 = jnp.concatenate([off, jnp.full((L - 9,), N, jnp.int32)])
    c1, c2 = _sc_counts(xs, ys, zs, b32, off16)
    feats = jnp.concatenate([x, pos, c1[:, None], c2[:, None]], axis=1)
    return feats, pos, batch


# speedup vs baseline: 2.6012x; 1.1530x over previous
import functools

import jax
import jax.numpy as jnp
import numpy as np
from jax import lax
from jax.experimental import pallas as pl
from jax.experimental.pallas import tpu as pltpu
from jax.experimental.pallas import tpu_sc as plsc

N = 8192
NC, NS, L = 2, 16, 16
NW = NC * NS
QPW = N // NW
CHUNKS = QPW // L
CPG = 4

T1 = np.float32(0.2 * 0.2)
T2 = np.float32(0.4 * 0.4)


def _sc_counts(xs, ys, zs, b32, off16):
    mesh = plsc.VectorSubcoreMesh(
        core_axis_name="c", subcore_axis_name="s",
        num_cores=NC, num_subcores=NS)

    @functools.partial(
        pl.kernel,
        out_type=(jax.ShapeDtypeStruct((N,), jnp.float32),
                  jax.ShapeDtypeStruct((N,), jnp.float32)),
        mesh=mesh,
        scratch_types=[
            pltpu.VMEM((N,), jnp.float32),
            pltpu.VMEM((N,), jnp.float32),
            pltpu.VMEM((N,), jnp.float32),
            pltpu.VMEM((N,), jnp.int32),
            pltpu.VMEM((L,), jnp.int32),
            pltpu.VMEM((N,), jnp.float32),
            pltpu.VMEM((QPW,), jnp.float32),
            pltpu.VMEM((QPW,), jnp.float32),
        ],
        compiler_params=pltpu.CompilerParams(
            use_tc_tiling_on_sc=False, needs_layout_passes=False),
    )
    def k(xs_h, ys_h, zs_h, b_h, off_h, c1_h, c2_h,
          xs_v, ys_v, zs_v, b_v, off_v, hw_v, c1_v, c2_v):
        wid = lax.axis_index("s") * NC + lax.axis_index("c")
        pltpu.sync_copy(xs_h, xs_v)
        pltpu.sync_copy(ys_h, ys_v)
        pltpu.sync_copy(zs_h, zs_v)
        pltpu.sync_copy(b_h, b_v)
        pltpu.sync_copy(off_h, off_v)
        qbase = wid * QPW
        lane = lax.iota(jnp.int32, L)

        def hw_body(i, _):
            idx = i * L + lane
            xv = plsc.load_gather(xs_v, [idx])
            yv = plsc.load_gather(ys_v, [idx])
            zv = plsc.load_gather(zs_v, [idx])
            plsc.store_scatter(
                hw_v, [idx],
                (xv * xv + yv * yv + zv * zv) * np.float32(0.5))
            return 0
        lax.fori_loop(0, N // L, hw_body, 0)

        for sc in range(QPW // (L * CPG)):
            qx, qy, qz, bq, ht1, ht2 = [], [], [], [], [], []
            jstart = jnp.int32(N)
            jend = jnp.int32(0)
            bmin = jnp.int32(127)
            bmax = jnp.int32(-1)
            for g in range(CPG):
                qidx = qbase + (sc * CPG + g) * L + lane
                qx.append(plsc.load_gather(xs_v, [qidx]))
                qy.append(plsc.load_gather(ys_v, [qidx]))
                qz.append(plsc.load_gather(zs_v, [qidx]))
                bq.append(plsc.load_gather(b_v, [qidx]))
                qn = qx[g] * qx[g] + qy[g] * qy[g] + qz[g] * qz[g]
                ht1.append((T1 - qn) * np.float32(0.5))
                ht2.append((T2 - qn) * np.float32(0.5))
                sv = plsc.load_gather(off_v, [bq[g]])
                ev = plsc.load_gather(off_v, [bq[g] + 1])
                jstart = jnp.minimum(jstart, jnp.min(sv))
                jend = jnp.maximum(jend, jnp.max(ev))
                bmin = jnp.minimum(bmin, jnp.min(bq[g]))
                bmax = jnp.maximum(bmax, jnp.max(bq[g]))

            both = jnp.int32(0x10001)
            one_i = jnp.int32(1)
            zero_i = jnp.int32(0)
            z16 = jnp.zeros((L,), jnp.int32)

            def make_body(masked):
                def body(j, carry):
                    jv = jnp.full((L,), j, dtype=jnp.int32)
                    xj = plsc.load_gather(xs_v, [jv])
                    yj = plsc.load_gather(ys_v, [jv])
                    zj = plsc.load_gather(zs_v, [jv])
                    hwj = plsc.load_gather(hw_v, [jv])
                    bj = plsc.load_gather(b_v, [jv]) if masked else None
                    out = []
                    for g in range(CPG):
                        u = hwj - (qx[g] * xj + qy[g] * yj + qz[g] * zj)
                        m1 = u <= ht1[g]
                        m2 = u <= ht2[g]
                        if masked:
                            m2 = m2 & (bj == bq[g])
                        step = jnp.where(m2, jnp.where(m1, both, one_i), zero_i)
                        out.append(carry[g] + step)
                    return tuple(out)
                return body

            def run(masked):
                def f(_):
                    return plsc.parallel_loop(
                        jstart, jend, 1, unroll=2,
                        carry=(z16,) * CPG)(make_body(masked))
                return f

            acc = lax.cond(bmin == bmax, run(False), run(True), 0)
            for g in range(CPG):
                o = (sc * CPG + g) * L
                c1 = (acc[g] >> 16).astype(jnp.float32)
                c2 = (acc[g] & jnp.int32(0xFFFF)).astype(jnp.float32)
                c1_v[pl.ds(o, L)] = (
                    jnp.minimum(c1, np.float32(32.0)) * np.float32(1.0 / 32.0))
                c2_v[pl.ds(o, L)] = (
                    jnp.minimum(c2, np.float32(64.0)) * np.float32(1.0 / 64.0))

        pltpu.sync_copy(c1_v, c1_h.at[pl.ds(qbase, QPW)])
        pltpu.sync_copy(c2_v, c2_h.at[pl.ds(qbase, QPW)])

    return k(xs, ys, zs, b32, off16)


def kernel(x, pos, batch):
    pos = pos.astype(jnp.float32)
    xs = pos[:, 0]
    ys = pos[:, 1]
    zs = pos[:, 2]
    b32 = batch.astype(jnp.int32)
    off = jnp.searchsorted(b32, jnp.arange(9, dtype=jnp.int32)).astype(jnp.int32)
    off16 = jnp.concatenate([off, jnp.full((L - 9,), N, jnp.int32)])
    c1, c2 = _sc_counts(xs, ys, zs, b32, off16)
    feats = jnp.concatenate([x, pos, c1[:, None], c2[:, None]], axis=1)
    return feats, pos, batch


# speedup vs baseline: 2.9991x; 1.1530x over previous
import functools

import jax
import jax.numpy as jnp
import numpy as np
from jax import lax
from jax.experimental import pallas as pl
from jax.experimental.pallas import tpu as pltpu
from jax.experimental.pallas import tpu_sc as plsc

N = 8192
NC, NS, L = 2, 16, 16
NW = NC * NS
QPW = N // NW
CHUNKS = QPW // L
CPG = 4

T1 = np.float32(0.2 * 0.2)
T2 = np.float32(0.4 * 0.4)


def _sc_counts(xs, ys, zs, b32, off16):
    mesh = plsc.VectorSubcoreMesh(
        core_axis_name="c", subcore_axis_name="s",
        num_cores=NC, num_subcores=NS)

    @functools.partial(
        pl.kernel,
        out_type=(jax.ShapeDtypeStruct((N,), jnp.float32),
                  jax.ShapeDtypeStruct((N,), jnp.float32)),
        mesh=mesh,
        scratch_types=[
            pltpu.VMEM((N,), jnp.float32),
            pltpu.VMEM((N,), jnp.float32),
            pltpu.VMEM((N,), jnp.float32),
            pltpu.VMEM((N,), jnp.int32),
            pltpu.VMEM((L,), jnp.int32),
            pltpu.VMEM((N,), jnp.float32),
            pltpu.VMEM((QPW,), jnp.float32),
            pltpu.VMEM((QPW,), jnp.float32),
        ],
        compiler_params=pltpu.CompilerParams(
            use_tc_tiling_on_sc=False, needs_layout_passes=False),
    )
    def k(xs_h, ys_h, zs_h, b_h, off_h, c1_h, c2_h,
          xs_v, ys_v, zs_v, b_v, off_v, hw_v, c1_v, c2_v):
        wid = lax.axis_index("s") * NC + lax.axis_index("c")
        pltpu.sync_copy(xs_h, xs_v)
        pltpu.sync_copy(ys_h, ys_v)
        pltpu.sync_copy(zs_h, zs_v)
        pltpu.sync_copy(b_h, b_v)
        pltpu.sync_copy(off_h, off_v)
        qbase = wid * QPW
        lane = lax.iota(jnp.int32, L)

        def hw_body(i, _):
            idx = i * L + lane
            xv = plsc.load_gather(xs_v, [idx])
            yv = plsc.load_gather(ys_v, [idx])
            zv = plsc.load_gather(zs_v, [idx])
            plsc.store_scatter(
                hw_v, [idx],
                (xv * xv + yv * yv + zv * zv) * np.float32(0.5))
            return 0
        lax.fori_loop(0, N // L, hw_body, 0)

        for sc in range(QPW // (L * CPG)):
            qx, qy, qz, bq, ht1, ht2 = [], [], [], [], [], []
            jstart = jnp.int32(N)
            jend = jnp.int32(0)
            bmin = jnp.int32(127)
            bmax = jnp.int32(-1)
            for g in range(CPG):
                qidx = qbase + (sc * CPG + g) * L + lane
                qx.append(plsc.load_gather(xs_v, [qidx]))
                qy.append(plsc.load_gather(ys_v, [qidx]))
                qz.append(plsc.load_gather(zs_v, [qidx]))
                bq.append(plsc.load_gather(b_v, [qidx]))
                qn = qx[g] * qx[g] + qy[g] * qy[g] + qz[g] * qz[g]
                ht1.append((T1 - qn) * np.float32(0.5))
                ht2.append((T2 - qn) * np.float32(0.5))
                sv = plsc.load_gather(off_v, [bq[g]])
                ev = plsc.load_gather(off_v, [bq[g] + 1])
                jstart = jnp.minimum(jstart, jnp.min(sv))
                jend = jnp.maximum(jend, jnp.max(ev))
                bmin = jnp.minimum(bmin, jnp.min(bq[g]))
                bmax = jnp.maximum(bmax, jnp.max(bq[g]))

            both = jnp.int32(0x10001)
            one_i = jnp.int32(1)
            zero_i = jnp.int32(0)
            z16 = jnp.zeros((L,), jnp.int32)

            def make_body(masked):
                def body(j, carry):
                    jv = jnp.full((L,), j, dtype=jnp.int32)
                    xj = plsc.load_gather(xs_v, [jv])
                    yj = plsc.load_gather(ys_v, [jv])
                    zj = plsc.load_gather(zs_v, [jv])
                    hwj = plsc.load_gather(hw_v, [jv])
                    bj = plsc.load_gather(b_v, [jv]) if masked else None
                    out = []
                    for g in range(CPG):
                        u = hwj - (qx[g] * xj + qy[g] * yj + qz[g] * zj)
                        m1 = u <= ht1[g]
                        m2 = u <= ht2[g]
                        if masked:
                            m2 = m2 & (bj == bq[g])
                        step = jnp.where(m2, jnp.where(m1, both, one_i), zero_i)
                        out.append(carry[g] + step)
                    return tuple(out)
                return body

            def run(masked):
                def f(_):
                    return plsc.parallel_loop(
                        jstart, jend, 1, unroll=2,
                        carry=(z16,) * CPG)(make_body(masked))
                return f

            acc = lax.cond(bmin == bmax, run(False), run(True), 0)
            for g in range(CPG):
                o = (sc * CPG + g) * L
                c1 = (acc[g] >> 16).astype(jnp.float32)
                c2 = (acc[g] & jnp.int32(0xFFFF)).astype(jnp.float32)
                c1_v[pl.ds(o, L)] = (
                    jnp.minimum(c1, np.float32(32.0)) * np.float32(1.0 / 32.0))
                c2_v[pl.ds(o, L)] = (
                    jnp.minimum(c2, np.float32(64.0)) * np.float32(1.0 / 64.0))

        pltpu.sync_copy(c1_v, c1_h.at[pl.ds(qbase, QPW)])
        pltpu.sync_copy(c2_v, c2_h.at[pl.ds(qbase, QPW)])

    return k(xs, ys, zs, b32, off16)


def kernel(x, pos, batch):
    pos = pos.astype(jnp.float32)
    xs = pos[:, 0]
    ys = pos[:, 1]
    zs = pos[:, 2]
    b32 = batch.astype(jnp.int32)
    off16 = jnp.sum(b32[:, None] < jnp.arange(L, dtype=jnp.int32)[None, :],
                    axis=0, dtype=jnp.int32)
    c1, c2 = _sc_counts(xs, ys, zs, b32, off16)
    feats = jnp.concatenate([x, pos, c1[:, None], c2[:, None]], axis=1)
    return feats, pos, batch


# speedup vs baseline: 3.0723x; 1.0244x over previous
import functools

import jax
import jax.numpy as jnp
import numpy as np
from jax import lax
from jax.experimental import pallas as pl
from jax.experimental.pallas import tpu as pltpu
from jax.experimental.pallas import tpu_sc as plsc

N = 8192
NC, NS, L = 2, 16, 16
NW = NC * NS
QPW = N // NW
CHUNKS = QPW // L
CPG = 4

T1 = np.float32(0.2 * 0.2)
T2 = np.float32(0.4 * 0.4)


def _sc_counts(xs, ys, zs, b32, off16):
    mesh = plsc.VectorSubcoreMesh(
        core_axis_name="c", subcore_axis_name="s",
        num_cores=NC, num_subcores=NS)

    @functools.partial(
        pl.kernel,
        out_type=(jax.ShapeDtypeStruct((N,), jnp.float32),
                  jax.ShapeDtypeStruct((N,), jnp.float32)),
        mesh=mesh,
        scratch_types=[
            pltpu.VMEM((N,), jnp.float32),
            pltpu.VMEM((N,), jnp.float32),
            pltpu.VMEM((N,), jnp.float32),
            pltpu.VMEM((N,), jnp.int32),
            pltpu.VMEM((L,), jnp.int32),
            pltpu.VMEM((N,), jnp.float32),
            pltpu.VMEM((QPW,), jnp.float32),
            pltpu.VMEM((QPW,), jnp.float32),
            pltpu.SemaphoreType.DMA,
        ],
        compiler_params=pltpu.CompilerParams(
            use_tc_tiling_on_sc=False, needs_layout_passes=False),
    )
    def k(xs_h, ys_h, zs_h, b_h, off_h, c1_h, c2_h,
          xs_v, ys_v, zs_v, b_v, off_v, hw_v, c1_v, c2_v, dsem):
        wid = lax.axis_index("s") * NC + lax.axis_index("c")
        cps = [pltpu.async_copy(s, d, dsem)
               for s, d in ((xs_h, xs_v), (ys_h, ys_v), (zs_h, zs_v),
                            (b_h, b_v), (off_h, off_v))]
        for cp in cps:
            cp.wait()
        qbase = wid * QPW
        lane = lax.iota(jnp.int32, L)

        def hw_body(i, _):
            idx = i * L + lane
            xv = plsc.load_gather(xs_v, [idx])
            yv = plsc.load_gather(ys_v, [idx])
            zv = plsc.load_gather(zs_v, [idx])
            plsc.store_scatter(
                hw_v, [idx],
                (xv * xv + yv * yv + zv * zv) * np.float32(0.5))
            return 0
        lax.fori_loop(0, N // L, hw_body, 0)

        for sc in range(QPW // (L * CPG)):
            qx, qy, qz, bq, ht1, ht2 = [], [], [], [], [], []
            jstart = jnp.int32(N)
            jend = jnp.int32(0)
            bmin = jnp.int32(127)
            bmax = jnp.int32(-1)
            for g in range(CPG):
                qidx = qbase + (sc * CPG + g) * L + lane
                qx.append(plsc.load_gather(xs_v, [qidx]))
                qy.append(plsc.load_gather(ys_v, [qidx]))
                qz.append(plsc.load_gather(zs_v, [qidx]))
                bq.append(plsc.load_gather(b_v, [qidx]))
                qn = qx[g] * qx[g] + qy[g] * qy[g] + qz[g] * qz[g]
                ht1.append((T1 - qn) * np.float32(0.5))
                ht2.append((T2 - qn) * np.float32(0.5))
                sv = plsc.load_gather(off_v, [bq[g]])
                ev = plsc.load_gather(off_v, [bq[g] + 1])
                jstart = jnp.minimum(jstart, jnp.min(sv))
                jend = jnp.maximum(jend, jnp.max(ev))
                bmin = jnp.minimum(bmin, jnp.min(bq[g]))
                bmax = jnp.maximum(bmax, jnp.max(bq[g]))

            both = jnp.int32(0x10001)
            one_i = jnp.int32(1)
            zero_i = jnp.int32(0)
            z16 = jnp.zeros((L,), jnp.int32)

            def make_body(masked):
                def body(j, carry):
                    jv = jnp.full((L,), j, dtype=jnp.int32)
                    xj = plsc.load_gather(xs_v, [jv])
                    yj = plsc.load_gather(ys_v, [jv])
                    zj = plsc.load_gather(zs_v, [jv])
                    hwj = plsc.load_gather(hw_v, [jv])
                    bj = plsc.load_gather(b_v, [jv]) if masked else None
                    out = []
                    for g in range(CPG):
                        u = hwj - (qx[g] * xj + qy[g] * yj + qz[g] * zj)
                        m1 = u <= ht1[g]
                        m2 = u <= ht2[g]
                        if masked:
                            m2 = m2 & (bj == bq[g])
                        step = jnp.where(m2, jnp.where(m1, both, one_i), zero_i)
                        out.append(carry[g] + step)
                    return tuple(out)
                return body

            def run(masked):
                def f(_):
                    return plsc.parallel_loop(
                        jstart, jend, 1, unroll=2,
                        carry=(z16,) * CPG)(make_body(masked))
                return f

            acc = lax.cond(bmin == bmax, run(False), run(True), 0)
            for g in range(CPG):
                o = (sc * CPG + g) * L
                c1 = (acc[g] >> 16).astype(jnp.float32)
                c2 = (acc[g] & jnp.int32(0xFFFF)).astype(jnp.float32)
                c1_v[pl.ds(o, L)] = (
                    jnp.minimum(c1, np.float32(32.0)) * np.float32(1.0 / 32.0))
                c2_v[pl.ds(o, L)] = (
                    jnp.minimum(c2, np.float32(64.0)) * np.float32(1.0 / 64.0))

        pltpu.sync_copy(c1_v, c1_h.at[pl.ds(qbase, QPW)])
        pltpu.sync_copy(c2_v, c2_h.at[pl.ds(qbase, QPW)])

    return k(xs, ys, zs, b32, off16)


def kernel(x, pos, batch):
    pos = pos.astype(jnp.float32)
    xs = pos[:, 0]
    ys = pos[:, 1]
    zs = pos[:, 2]
    b32 = batch.astype(jnp.int32)
    off16 = jnp.sum(b32[:, None] < jnp.arange(L, dtype=jnp.int32)[None, :],
                    axis=0, dtype=jnp.int32)
    c1, c2 = _sc_counts(xs, ys, zs, b32, off16)
    feats = jnp.concatenate([x, pos, c1[:, None], c2[:, None]], axis=1)
    return feats, pos, batch
